# baseline (speedup 1.0000x reference)
import functools

import jax
import jax.numpy as jnp
from jax import lax
from jax.experimental import pallas as pl
from jax.experimental.pallas import tpu as pltpu
from jax.experimental.pallas import tpu_sc as plsc

EMBED = 64
LANES = 16
BCOLS = 128
VBLK = 384
NSLOT = 2
NW = 32


def _pairs_body(table_t_hbm, tail_hbm, pairs_hbm, src_v, dst_v, tail_v,
                ssem0, ssem1, wsem0, wsem1, vocab_main):
    ssems = (ssem0, ssem1)
    wsems = (wsem0, wsem1)
    nc = 2
    wid = lax.axis_index("s") * nc + lax.axis_index("c")
    nblk_total = vocab_main // VBLK
    nblk_w = (nblk_total - wid + NW - 1) // NW

    i16 = lax.iota(jnp.int32, LANES)
    cpats = [(i16 + d) & (LANES - 1) for d in range(LANES)]

    def blk_col(k):
        return (wid + NW * k) * VBLK

    def s_start(k, b):
        pltpu.async_copy(
            table_t_hbm.at[:, pl.ds(pl.multiple_of(blk_col(k), 128), VBLK)],
            src_v.at[b], ssems[b])

    def s_wait(b):
        pltpu.make_async_copy(table_t_hbm.at[:, pl.ds(0, VBLK)],
                              src_v.at[b], ssems[b]).wait()

    def wb_start(k, b):
        pltpu.async_copy(
            dst_v.at[b],
            pairs_hbm.at[pl.ds(pl.multiple_of(blk_col(k) // 2, 8),
                               VBLK // 2)],
            wsems[b])

    def wb_wait(b):
        pltpu.make_async_copy(dst_v.at[b],
                              pairs_hbm.at[pl.ds(0, VBLK // 2)],
                              wsems[b]).wait()

    h64p = (i16 & 1) * EMBED

    def transpose(b):
        src = src_v.at[b]
        dst = dst_v.at[b]

        def v_body(vb, carry):
            vvec = i16 + vb * LANES
            rowv = lax.shift_right_logical(i16, 1) + vb * (LANES // 2)

            def e_body(k, carry2, vvec=vvec, rowv=rowv):
                ebase = k * LANES
                hb = h64p + ebase
                for d in range(LANES):
                    val = plsc.load_gather(src, [ebase + cpats[d], vvec])
                    plsc.store_scatter(dst, [rowv, hb + cpats[d]], val)
                return carry2

            lax.fori_loop(0, EMBED // LANES, e_body, 0)
            return carry

        lax.fori_loop(0, VBLK // LANES, v_body, 0)

    @pl.when(wid == 0)
    def _():
        pltpu.sync_copy(tail_hbm, tail_v)
        pltpu.sync_copy(tail_v, pairs_hbm.at[pl.ds(vocab_main // 2,
                                                   tail_v.shape[0])])

    pl.when(nblk_w > 0)(lambda: s_start(0, 0))
    pl.when(nblk_w > 1)(lambda: s_start(1, 1))

    def blk_body(p, carry):
        for b in range(NSLOT):
            k = NSLOT * p + b

            @pl.when(k < nblk_w)
            def _(p=p, k=k, b=b):
                s_wait(b)
                pl.when(p > 0)(lambda b=b: wb_wait(b))
                transpose(b)
                pl.when(k + NSLOT < nblk_w)(
                    lambda k=k, b=b: s_start(k + NSLOT, b))
                wb_start(k, b)
        return carry

    lax.fori_loop(0, (nblk_w + NSLOT - 1) // NSLOT, blk_body, 0)

    for b in range(NSLOT):
        pl.when(nblk_w > b)(lambda b=b: wb_wait(b))


def _emb_body(tok_hbm, pairs_hbm, out_hbm, idx_v, pidx_v, h64_v,
              rows_v, tr_v, gsem0, gsem1, wsem0, wsem1, seq):
    gsems = (gsem0, gsem1)
    wsems = (wsem0, wsem1)
    nc = 2
    wid = lax.axis_index("s") * nc + lax.axis_index("c")
    col0 = wid * BCOLS
    n_pairs = seq // NSLOT

    i16 = lax.iota(jnp.int32, LANES)
    cpats = [(i16 + d) & (LANES - 1) for d in range(LANES)]

    pltpu.sync_copy(tok_hbm.at[:, pl.ds(col0, BCOLS)], idx_v)

    def prep_body(s, carry):
        for bb in range(BCOLS // LANES):
            v = idx_v[s, pl.ds(bb * LANES, LANES)]
            pidx_v[s, pl.ds(bb * LANES, LANES)] = lax.shift_right_logical(v, 1)
            h64_v[s, pl.ds(bb * LANES, LANES)] = lax.shift_left(v & 1, 6)
        return carry

    lax.fori_loop(0, seq, prep_body, 0)

    def g_start(s, b):
        pltpu.async_copy(pairs_hbm.at[pidx_v.at[s]], rows_v.at[b], gsems[b])

    def g_wait(b):
        pltpu.make_async_copy(pairs_hbm.at[pidx_v.at[0]], rows_v.at[b],
                              gsems[b]).wait()

    def wb_start(s, b):
        pltpu.async_copy(tr_v.at[b], out_hbm.at[s, :, pl.ds(col0, BCOLS)],
                         wsems[b])

    def wb_wait(b):
        pltpu.make_async_copy(tr_v.at[b], out_hbm.at[0, :, pl.ds(col0, BCOLS)],
                              wsems[b]).wait()

    def transpose(s, b):
        rows = rows_v.at[b]
        tr = tr_v.at[b]

        def bb_body(bb, carry):
            rvec = i16 + bb * LANES
            hv = h64_v[s, pl.ds(bb * LANES, LANES)]

            def e_body(k, carry2, rvec=rvec, hv=hv):
                ebase = k * LANES
                hbase = ebase + hv
                for d in range(LANES):
                    evec = ebase + cpats[d]
                    cvec = hbase + cpats[d]
                    val = plsc.load_gather(rows, [rvec, cvec])
                    plsc.store_scatter(tr, [evec, rvec], val)
                return carry2

            lax.fori_loop(0, EMBED // LANES, e_body, 0)
            return carry

        lax.fori_loop(0, BCOLS // LANES, bb_body, 0)

    g_start(0, 0)
    g_start(1, 1)

    def pair_body(p, carry):
        for b in range(NSLOT):
            s = NSLOT * p + b
            g_wait(b)
            pl.when(p > 0)(lambda b=b: wb_wait(b))
            transpose(s, b)
            pl.when(p < n_pairs - 1)(lambda s=s, b=b: g_start(s + NSLOT, b))
            wb_start(s, b)
        return carry

    lax.fori_loop(0, n_pairs, pair_body, 0)

    for b in range(NSLOT):
        wb_wait(b)


def kernel(tokens_ids, table):
    batch, seq = tokens_ids.shape
    vocab, embed = table.shape
    assert embed == EMBED and batch == NW * BCOLS
    assert seq % NSLOT == 0 and seq >= 2 * NSLOT
    vocab_main = (vocab // VBLK) * VBLK
    tail_rows = (vocab - vocab_main) // 2
    assert (vocab - vocab_main) % 2 == 0

    mesh = plsc.VectorSubcoreMesh(core_axis_name="c", subcore_axis_name="s")
    params = pltpu.CompilerParams(use_tc_tiling_on_sc=True,
                                  needs_layout_passes=False)

    tok_t = tokens_ids.T
    table_t = table.T
    tail = jnp.reshape(table[vocab_main:, :], (tail_rows, 2 * EMBED))

    pairs_kernel = pl.kernel(
        functools.partial(_pairs_body, vocab_main=vocab_main),
        out_type=jax.ShapeDtypeStruct((vocab // 2, 2 * EMBED), jnp.float32),
        mesh=mesh,
        scratch_types=[
            pltpu.VMEM((NSLOT, EMBED, VBLK), jnp.float32),
            pltpu.VMEM((NSLOT, VBLK // 2, 2 * EMBED), jnp.float32),
            pltpu.VMEM((tail_rows, 2 * EMBED), jnp.float32),
            pltpu.SemaphoreType.DMA,
            pltpu.SemaphoreType.DMA,
            pltpu.SemaphoreType.DMA,
            pltpu.SemaphoreType.DMA,
        ],
        compiler_params=params,
    )
    table_pairs = pairs_kernel(table_t, tail)

    emb_kernel = pl.kernel(
        functools.partial(_emb_body, seq=seq),
        out_type=jax.ShapeDtypeStruct((seq, embed, batch), jnp.float32),
        mesh=mesh,
        scratch_types=[
            pltpu.VMEM((seq, BCOLS), jnp.int32),
            pltpu.VMEM((seq, BCOLS), jnp.int32),
            pltpu.VMEM((seq, BCOLS), jnp.int32),
            pltpu.VMEM((NSLOT, BCOLS, 2 * EMBED), jnp.float32),
            pltpu.VMEM((NSLOT, EMBED, BCOLS), jnp.float32),
            pltpu.SemaphoreType.DMA,
            pltpu.SemaphoreType.DMA,
            pltpu.SemaphoreType.DMA,
            pltpu.SemaphoreType.DMA,
        ],
        compiler_params=params,
    )
    out_t = emb_kernel(tok_t, table_pairs)
    return jnp.transpose(out_t, (2, 0, 1))


# speedup vs baseline: 1.1409x; 1.1409x over previous
import functools

import jax
import jax.numpy as jnp
from jax import lax
from jax.experimental import pallas as pl
from jax.experimental.pallas import tpu as pltpu
from jax.experimental.pallas import tpu_sc as plsc

EMBED = 64
LANES = 16
BCOLS = 128
VBLK = 384
NSLOT = 2
NW = 32


def _pairs_body(table_t_hbm, tail_hbm, pairs_hbm, src_v, dst_v, tail_v,
                ssem0, ssem1, wsem0, wsem1, vocab_main):
    ssems = (ssem0, ssem1)
    wsems = (wsem0, wsem1)
    nc = 2
    wid = lax.axis_index("s") * nc + lax.axis_index("c")
    nblk_total = vocab_main // VBLK
    nblk_w = (nblk_total - wid + NW - 1) // NW

    i16 = lax.iota(jnp.int32, LANES)
    cpats = [(i16 + d) & (LANES - 1) for d in range(LANES)]

    def blk_col(k):
        return (wid + NW * k) * VBLK

    def s_start(k, b):
        pltpu.async_copy(
            table_t_hbm.at[:, pl.ds(pl.multiple_of(blk_col(k), 128), VBLK)],
            src_v.at[b], ssems[b])

    def s_wait(b):
        pltpu.make_async_copy(table_t_hbm.at[:, pl.ds(0, VBLK)],
                              src_v.at[b], ssems[b]).wait()

    def wb_start(k, b):
        pltpu.async_copy(
            dst_v.at[b],
            pairs_hbm.at[pl.ds(pl.multiple_of(blk_col(k) // 2, 8),
                               VBLK // 2)],
            wsems[b])

    def wb_wait(b):
        pltpu.make_async_copy(dst_v.at[b],
                              pairs_hbm.at[pl.ds(0, VBLK // 2)],
                              wsems[b]).wait()

    h64p = (i16 & 1) * EMBED

    def transpose(b):
        src = src_v.at[b]
        dst = dst_v.at[b]

        @plsc.parallel_loop(0, VBLK // LANES, unroll=2)
        def _(vb):
            vvec = i16 + vb * LANES
            rowv = lax.shift_right_logical(i16, 1) + vb * (LANES // 2)

            def e_body(k, carry2, vvec=vvec, rowv=rowv):
                ebase = k * LANES
                hb = h64p + ebase
                for d in range(LANES):
                    val = plsc.load_gather(src, [ebase + cpats[d], vvec])
                    plsc.store_scatter(dst, [rowv, hb + cpats[d]], val)
                return carry2

            lax.fori_loop(0, EMBED // LANES, e_body, 0)

    @pl.when(wid == 0)
    def _():
        pltpu.sync_copy(tail_hbm, tail_v)
        pltpu.sync_copy(tail_v, pairs_hbm.at[pl.ds(vocab_main // 2,
                                                   tail_v.shape[0])])

    pl.when(nblk_w > 0)(lambda: s_start(0, 0))
    pl.when(nblk_w > 1)(lambda: s_start(1, 1))

    def blk_body(p, carry):
        for b in range(NSLOT):
            k = NSLOT * p + b

            @pl.when(k < nblk_w)
            def _(p=p, k=k, b=b):
                s_wait(b)
                pl.when(p > 0)(lambda b=b: wb_wait(b))
                transpose(b)
                pl.when(k + NSLOT < nblk_w)(
                    lambda k=k, b=b: s_start(k + NSLOT, b))
                wb_start(k, b)
        return carry

    lax.fori_loop(0, (nblk_w + NSLOT - 1) // NSLOT, blk_body, 0)

    for b in range(NSLOT):
        pl.when(nblk_w > b)(lambda b=b: wb_wait(b))


def _emb_body(tok_hbm, pairs_hbm, out_hbm, idx_v, pidx_v, h64_v,
              rows_v, tr_v, gsem0, gsem1, wsem0, wsem1, seq):
    gsems = (gsem0, gsem1)
    wsems = (wsem0, wsem1)
    nc = 2
    wid = lax.axis_index("s") * nc + lax.axis_index("c")
    col0 = wid * BCOLS
    n_pairs = seq // NSLOT

    i16 = lax.iota(jnp.int32, LANES)
    cpats = [(i16 + d) & (LANES - 1) for d in range(LANES)]

    pltpu.sync_copy(tok_hbm.at[:, pl.ds(col0, BCOLS)], idx_v)

    def prep_body(s, carry):
        for bb in range(BCOLS // LANES):
            v = idx_v[s, pl.ds(bb * LANES, LANES)]
            pidx_v[s, pl.ds(bb * LANES, LANES)] = lax.shift_right_logical(v, 1)
            h64_v[s, pl.ds(bb * LANES, LANES)] = lax.shift_left(v & 1, 6)
        return carry

    lax.fori_loop(0, seq, prep_body, 0)

    def g_start(s, b):
        pltpu.async_copy(pairs_hbm.at[pidx_v.at[s]], rows_v.at[b], gsems[b])

    def g_wait(b):
        pltpu.make_async_copy(pairs_hbm.at[pidx_v.at[0]], rows_v.at[b],
                              gsems[b]).wait()

    def wb_start(s, b):
        pltpu.async_copy(tr_v.at[b], out_hbm.at[s, :, pl.ds(col0, BCOLS)],
                         wsems[b])

    def wb_wait(b):
        pltpu.make_async_copy(tr_v.at[b], out_hbm.at[0, :, pl.ds(col0, BCOLS)],
                              wsems[b]).wait()

    def transpose(s, b):
        rows = rows_v.at[b]
        tr = tr_v.at[b]

        @plsc.parallel_loop(0, BCOLS // LANES, unroll=2)
        def _(bb):
            rvec = i16 + bb * LANES
            hv = h64_v[s, pl.ds(bb * LANES, LANES)]

            def e_body(k, carry2, rvec=rvec, hv=hv):
                ebase = k * LANES
                hbase = ebase + hv
                for d in range(LANES):
                    evec = ebase + cpats[d]
                    cvec = hbase + cpats[d]
                    val = plsc.load_gather(rows, [rvec, cvec])
                    plsc.store_scatter(tr, [evec, rvec], val)
                return carry2

            lax.fori_loop(0, EMBED // LANES, e_body, 0)

    g_start(0, 0)
    g_start(1, 1)

    def pair_body(p, carry):
        for b in range(NSLOT):
            s = NSLOT * p + b
            g_wait(b)
            pl.when(p > 0)(lambda b=b: wb_wait(b))
            transpose(s, b)
            pl.when(p < n_pairs - 1)(lambda s=s, b=b: g_start(s + NSLOT, b))
            wb_start(s, b)
        return carry

    lax.fori_loop(0, n_pairs, pair_body, 0)

    for b in range(NSLOT):
        wb_wait(b)


def kernel(tokens_ids, table):
    batch, seq = tokens_ids.shape
    vocab, embed = table.shape
    assert embed == EMBED and batch == NW * BCOLS
    assert seq % NSLOT == 0 and seq >= 2 * NSLOT
    vocab_main = (vocab // VBLK) * VBLK
    tail_rows = (vocab - vocab_main) // 2
    assert (vocab - vocab_main) % 2 == 0

    mesh = plsc.VectorSubcoreMesh(core_axis_name="c", subcore_axis_name="s")
    params = pltpu.CompilerParams(use_tc_tiling_on_sc=True,
                                  needs_layout_passes=False)

    tok_t = tokens_ids.T
    table_t = table.T
    tail = jnp.reshape(table[vocab_main:, :], (tail_rows, 2 * EMBED))

    pairs_kernel = pl.kernel(
        functools.partial(_pairs_body, vocab_main=vocab_main),
        out_type=jax.ShapeDtypeStruct((vocab // 2, 2 * EMBED), jnp.float32),
        mesh=mesh,
        scratch_types=[
            pltpu.VMEM((NSLOT, EMBED, VBLK), jnp.float32),
            pltpu.VMEM((NSLOT, VBLK // 2, 2 * EMBED), jnp.float32),
            pltpu.VMEM((tail_rows, 2 * EMBED), jnp.float32),
            pltpu.SemaphoreType.DMA,
            pltpu.SemaphoreType.DMA,
            pltpu.SemaphoreType.DMA,
            pltpu.SemaphoreType.DMA,
        ],
        compiler_params=params,
    )
    table_pairs = pairs_kernel(table_t, tail)

    emb_kernel = pl.kernel(
        functools.partial(_emb_body, seq=seq),
        out_type=jax.ShapeDtypeStruct((seq, embed, batch), jnp.float32),
        mesh=mesh,
        scratch_types=[
            pltpu.VMEM((seq, BCOLS), jnp.int32),
            pltpu.VMEM((seq, BCOLS), jnp.int32),
            pltpu.VMEM((seq, BCOLS), jnp.int32),
            pltpu.VMEM((NSLOT, BCOLS, 2 * EMBED), jnp.float32),
            pltpu.VMEM((NSLOT, EMBED, BCOLS), jnp.float32),
            pltpu.SemaphoreType.DMA,
            pltpu.SemaphoreType.DMA,
            pltpu.SemaphoreType.DMA,
            pltpu.SemaphoreType.DMA,
        ],
        compiler_params=params,
    )
    out_t = emb_kernel(tok_t, table_pairs)
    return jnp.transpose(out_t, (2, 0, 1))


# speedup vs baseline: 1.1782x; 1.0327x over previous
import functools

import jax
import jax.numpy as jnp
from jax import lax
from jax.experimental import pallas as pl
from jax.experimental.pallas import tpu as pltpu
from jax.experimental.pallas import tpu_sc as plsc

EMBED = 64
LANES = 16
BCOLS = 128
VBLK = 384
NSLOT = 2
NW = 32


def _pairs_body(table_t_hbm, tail_hbm, pairs_hbm, src_v, dst_v, tail_v,
                ssem0, ssem1, wsem0, wsem1, vocab_main):
    ssems = (ssem0, ssem1)
    wsems = (wsem0, wsem1)
    nc = 2
    wid = lax.axis_index("s") * nc + lax.axis_index("c")
    nblk_total = vocab_main // VBLK
    nblk_w = (nblk_total - wid + NW - 1) // NW

    i16 = lax.iota(jnp.int32, LANES)
    cpats = [(i16 + d) & (LANES - 1) for d in range(LANES)]

    def blk_col(k):
        return (wid + NW * k) * VBLK

    def s_start(k, b):
        pltpu.async_copy(
            table_t_hbm.at[:, pl.ds(pl.multiple_of(blk_col(k), 128), VBLK)],
            src_v.at[b], ssems[b])

    def s_wait(b):
        pltpu.make_async_copy(table_t_hbm.at[:, pl.ds(0, VBLK)],
                              src_v.at[b], ssems[b]).wait()

    def wb_start(k, b):
        pltpu.async_copy(
            dst_v.at[b],
            pairs_hbm.at[pl.ds(pl.multiple_of(blk_col(k) // 2, 8),
                               VBLK // 2)],
            wsems[b])

    def wb_wait(b):
        pltpu.make_async_copy(dst_v.at[b],
                              pairs_hbm.at[pl.ds(0, VBLK // 2)],
                              wsems[b]).wait()

    h64p = (i16 & 1) * EMBED

    def transpose(b):
        src = src_v.at[b]
        dst = dst_v.at[b]

        @plsc.parallel_loop(0, VBLK // LANES, unroll=4)
        def _(vb):
            vvec = i16 + vb * LANES
            rowv = lax.shift_right_logical(i16, 1) + vb * (LANES // 2)

            def e_body(k, carry2, vvec=vvec, rowv=rowv):
                ebase = k * LANES
                hb = h64p + ebase
                for d in range(LANES):
                    val = plsc.load_gather(src, [ebase + cpats[d], vvec])
                    plsc.store_scatter(dst, [rowv, hb + cpats[d]], val)
                return carry2

            lax.fori_loop(0, EMBED // LANES, e_body, 0)

    @pl.when(wid == 0)
    def _():
        pltpu.sync_copy(tail_hbm, tail_v)
        pltpu.sync_copy(tail_v, pairs_hbm.at[pl.ds(vocab_main // 2,
                                                   tail_v.shape[0])])

    pl.when(nblk_w > 0)(lambda: s_start(0, 0))
    pl.when(nblk_w > 1)(lambda: s_start(1, 1))

    def blk_body(p, carry):
        for b in range(NSLOT):
            k = NSLOT * p + b

            @pl.when(k < nblk_w)
            def _(p=p, k=k, b=b):
                s_wait(b)
                pl.when(p > 0)(lambda b=b: wb_wait(b))
                transpose(b)
                pl.when(k + NSLOT < nblk_w)(
                    lambda k=k, b=b: s_start(k + NSLOT, b))
                wb_start(k, b)
        return carry

    lax.fori_loop(0, (nblk_w + NSLOT - 1) // NSLOT, blk_body, 0)

    for b in range(NSLOT):
        pl.when(nblk_w > b)(lambda b=b: wb_wait(b))


def _emb_body(tok_hbm, pairs_hbm, out_hbm, idx_v, pidx_v, h64_v,
              rows_v, tr_v, gsem0, gsem1, wsem0, wsem1, seq):
    gsems = (gsem0, gsem1)
    wsems = (wsem0, wsem1)
    nc = 2
    wid = lax.axis_index("s") * nc + lax.axis_index("c")
    col0 = wid * BCOLS
    n_pairs = seq // NSLOT

    i16 = lax.iota(jnp.int32, LANES)
    cpats = [(i16 + d) & (LANES - 1) for d in range(LANES)]

    pltpu.sync_copy(tok_hbm.at[:, pl.ds(col0, BCOLS)], idx_v)

    def prep_body(s, carry):
        for bb in range(BCOLS // LANES):
            v = idx_v[s, pl.ds(bb * LANES, LANES)]
            pidx_v[s, pl.ds(bb * LANES, LANES)] = lax.shift_right_logical(v, 1)
            h64_v[s, pl.ds(bb * LANES, LANES)] = lax.shift_left(v & 1, 6)
        return carry

    lax.fori_loop(0, seq, prep_body, 0)

    def g_start(s, b):
        pltpu.async_copy(pairs_hbm.at[pidx_v.at[s]], rows_v.at[b], gsems[b])

    def g_wait(b):
        pltpu.make_async_copy(pairs_hbm.at[pidx_v.at[0]], rows_v.at[b],
                              gsems[b]).wait()

    def wb_start(s, b):
        pltpu.async_copy(tr_v.at[b], out_hbm.at[s, :, pl.ds(col0, BCOLS)],
                         wsems[b])

    def wb_wait(b):
        pltpu.make_async_copy(tr_v.at[b], out_hbm.at[0, :, pl.ds(col0, BCOLS)],
                              wsems[b]).wait()

    def transpose(s, b):
        rows = rows_v.at[b]
        tr = tr_v.at[b]

        @plsc.parallel_loop(0, BCOLS // LANES, unroll=4)
        def _(bb):
            rvec = i16 + bb * LANES
            hv = h64_v[s, pl.ds(bb * LANES, LANES)]

            def e_body(k, carry2, rvec=rvec, hv=hv):
                ebase = k * LANES
                hbase = ebase + hv
                for d in range(LANES):
                    evec = ebase + cpats[d]
                    cvec = hbase + cpats[d]
                    val = plsc.load_gather(rows, [rvec, cvec])
                    plsc.store_scatter(tr, [evec, rvec], val)
                return carry2

            lax.fori_loop(0, EMBED // LANES, e_body, 0)

    g_start(0, 0)
    g_start(1, 1)

    def pair_body(p, carry):
        for b in range(NSLOT):
            s = NSLOT * p + b
            g_wait(b)
            pl.when(p > 0)(lambda b=b: wb_wait(b))
            transpose(s, b)
            pl.when(p < n_pairs - 1)(lambda s=s, b=b: g_start(s + NSLOT, b))
            wb_start(s, b)
        return carry

    lax.fori_loop(0, n_pairs, pair_body, 0)

    for b in range(NSLOT):
        wb_wait(b)


def kernel(tokens_ids, table):
    batch, seq = tokens_ids.shape
    vocab, embed = table.shape
    assert embed == EMBED and batch == NW * BCOLS
    assert seq % NSLOT == 0 and seq >= 2 * NSLOT
    vocab_main = (vocab // VBLK) * VBLK
    tail_rows = (vocab - vocab_main) // 2
    assert (vocab - vocab_main) % 2 == 0

    mesh = plsc.VectorSubcoreMesh(core_axis_name="c", subcore_axis_name="s")
    params = pltpu.CompilerParams(use_tc_tiling_on_sc=True,
                                  needs_layout_passes=False)

    tok_t = tokens_ids.T
    table_t = table.T
    tail = jnp.reshape(table[vocab_main:, :], (tail_rows, 2 * EMBED))

    pairs_kernel = pl.kernel(
        functools.partial(_pairs_body, vocab_main=vocab_main),
        out_type=jax.ShapeDtypeStruct((vocab // 2, 2 * EMBED), jnp.float32),
        mesh=mesh,
        scratch_types=[
            pltpu.VMEM((NSLOT, EMBED, VBLK), jnp.float32),
            pltpu.VMEM((NSLOT, VBLK // 2, 2 * EMBED), jnp.float32),
            pltpu.VMEM((tail_rows, 2 * EMBED), jnp.float32),
            pltpu.SemaphoreType.DMA,
            pltpu.SemaphoreType.DMA,
            pltpu.SemaphoreType.DMA,
            pltpu.SemaphoreType.DMA,
        ],
        compiler_params=params,
    )
    table_pairs = pairs_kernel(table_t, tail)

    emb_kernel = pl.kernel(
        functools.partial(_emb_body, seq=seq),
        out_type=jax.ShapeDtypeStruct((seq, embed, batch), jnp.float32),
        mesh=mesh,
        scratch_types=[
            pltpu.VMEM((seq, BCOLS), jnp.int32),
            pltpu.VMEM((seq, BCOLS), jnp.int32),
            pltpu.VMEM((seq, BCOLS), jnp.int32),
            pltpu.VMEM((NSLOT, BCOLS, 2 * EMBED), jnp.float32),
            pltpu.VMEM((NSLOT, EMBED, BCOLS), jnp.float32),
            pltpu.SemaphoreType.DMA,
            pltpu.SemaphoreType.DMA,
            pltpu.SemaphoreType.DMA,
            pltpu.SemaphoreType.DMA,
        ],
        compiler_params=params,
    )
    out_t = emb_kernel(tok_t, table_pairs)
    return jnp.transpose(out_t, (2, 0, 1))


# speedup vs baseline: 2.0586x; 1.7473x over previous
import functools

import jax
import jax.numpy as jnp
from jax import lax
from jax.experimental import pallas as pl
from jax.experimental.pallas import tpu as pltpu
from jax.experimental.pallas import tpu_sc as plsc

EMBED = 64
LANES = 16
BCOLS = 128
VBLK = 384
NSLOT = 2
NW = 32


def _pairs_body(table_t_hbm, tail_hbm, pairs_hbm, src_v, dst_v, tail_v,
                ssem0, ssem1, wsem0, wsem1, vocab_main):
    ssems = (ssem0, ssem1)
    wsems = (wsem0, wsem1)
    nc = 2
    wid = lax.axis_index("s") * nc + lax.axis_index("c")
    nblk_total = vocab_main // VBLK
    nblk_w = (nblk_total - wid + NW - 1) // NW

    i16 = lax.iota(jnp.int32, LANES)
    cpats = [(i16 + d) & (LANES - 1) for d in range(LANES)]

    def blk_col(k):
        return (wid + NW * k) * VBLK

    def s_start(k, b):
        pltpu.async_copy(
            table_t_hbm.at[:, pl.ds(pl.multiple_of(blk_col(k), 128), VBLK)],
            src_v.at[b], ssems[b])

    def s_wait(b):
        pltpu.make_async_copy(table_t_hbm.at[:, pl.ds(0, VBLK)],
                              src_v.at[b], ssems[b]).wait()

    def wb_start(k, b):
        pltpu.async_copy(
            dst_v.at[b],
            pairs_hbm.at[pl.ds(pl.multiple_of(blk_col(k) // 2, 8),
                               VBLK // 2)],
            wsems[b])

    def wb_wait(b):
        pltpu.make_async_copy(dst_v.at[b],
                              pairs_hbm.at[pl.ds(0, VBLK // 2)],
                              wsems[b]).wait()

    h64p = (i16 & 1) * EMBED

    def transpose(b):
        src = src_v.at[b]
        dst = dst_v.at[b]

        @plsc.parallel_loop(0, VBLK // LANES, unroll=4)
        def _(vb):
            vvec = i16 + vb * LANES
            rowv = lax.shift_right_logical(i16, 1) + vb * (LANES // 2)
            for k in range(EMBED // LANES):
                ebase = k * LANES
                hb = h64p + ebase
                for d in range(LANES):
                    val = plsc.load_gather(src, [ebase + cpats[d], vvec])
                    plsc.store_scatter(dst, [rowv, hb + cpats[d]], val)

    @pl.when(wid == 0)
    def _():
        pltpu.sync_copy(tail_hbm, tail_v)
        pltpu.sync_copy(tail_v, pairs_hbm.at[pl.ds(vocab_main // 2,
                                                   tail_v.shape[0])])

    pl.when(nblk_w > 0)(lambda: s_start(0, 0))
    pl.when(nblk_w > 1)(lambda: s_start(1, 1))

    def blk_body(p, carry):
        for b in range(NSLOT):
            k = NSLOT * p + b

            @pl.when(k < nblk_w)
            def _(p=p, k=k, b=b):
                s_wait(b)
                pl.when(p > 0)(lambda b=b: wb_wait(b))
                transpose(b)
                pl.when(k + NSLOT < nblk_w)(
                    lambda k=k, b=b: s_start(k + NSLOT, b))
                wb_start(k, b)
        return carry

    lax.fori_loop(0, (nblk_w + NSLOT - 1) // NSLOT, blk_body, 0)

    for b in range(NSLOT):
        pl.when(nblk_w > b)(lambda b=b: wb_wait(b))


def _emb_body(tok_hbm, pairs_hbm, out_hbm, idx_v, pidx_v, h64_v,
              rows_v, tr_v, gsem0, gsem1, wsem0, wsem1, seq):
    gsems = (gsem0, gsem1)
    wsems = (wsem0, wsem1)
    nc = 2
    wid = lax.axis_index("s") * nc + lax.axis_index("c")
    col0 = wid * BCOLS
    n_pairs = seq // NSLOT

    i16 = lax.iota(jnp.int32, LANES)
    cpats = [(i16 + d) & (LANES - 1) for d in range(LANES)]

    pltpu.sync_copy(tok_hbm.at[:, pl.ds(col0, BCOLS)], idx_v)

    def prep_body(s, carry):
        for bb in range(BCOLS // LANES):
            v = idx_v[s, pl.ds(bb * LANES, LANES)]
            pidx_v[s, pl.ds(bb * LANES, LANES)] = lax.shift_right_logical(v, 1)
            h64_v[s, pl.ds(bb * LANES, LANES)] = lax.shift_left(v & 1, 6)
        return carry

    lax.fori_loop(0, seq, prep_body, 0)

    def g_start(s, b):
        pltpu.async_copy(pairs_hbm.at[pidx_v.at[s]], rows_v.at[b], gsems[b])

    def g_wait(b):
        pltpu.make_async_copy(pairs_hbm.at[pidx_v.at[0]], rows_v.at[b],
                              gsems[b]).wait()

    def wb_start(s, b):
        pltpu.async_copy(tr_v.at[b], out_hbm.at[s, :, pl.ds(col0, BCOLS)],
                         wsems[b])

    def wb_wait(b):
        pltpu.make_async_copy(tr_v.at[b], out_hbm.at[0, :, pl.ds(col0, BCOLS)],
                              wsems[b]).wait()

    def transpose(s, b):
        rows = rows_v.at[b]
        tr = tr_v.at[b]

        @plsc.parallel_loop(0, BCOLS // LANES, unroll=4)
        def _(bb):
            rvec = i16 + bb * LANES
            hv = h64_v[s, pl.ds(bb * LANES, LANES)]
            for k in range(EMBED // LANES):
                ebase = k * LANES
                hbase = ebase + hv
                for d in range(LANES):
                    evec = ebase + cpats[d]
                    cvec = hbase + cpats[d]
                    val = plsc.load_gather(rows, [rvec, cvec])
                    plsc.store_scatter(tr, [evec, rvec], val)

    g_start(0, 0)
    g_start(1, 1)

    def pair_body(p, carry):
        for b in range(NSLOT):
            s = NSLOT * p + b
            g_wait(b)
            pl.when(p > 0)(lambda b=b: wb_wait(b))
            transpose(s, b)
            pl.when(p < n_pairs - 1)(lambda s=s, b=b: g_start(s + NSLOT, b))
            wb_start(s, b)
        return carry

    lax.fori_loop(0, n_pairs, pair_body, 0)

    for b in range(NSLOT):
        wb_wait(b)


def kernel(tokens_ids, table):
    batch, seq = tokens_ids.shape
    vocab, embed = table.shape
    assert embed == EMBED and batch == NW * BCOLS
    assert seq % NSLOT == 0 and seq >= 2 * NSLOT
    vocab_main = (vocab // VBLK) * VBLK
    tail_rows = (vocab - vocab_main) // 2
    assert (vocab - vocab_main) % 2 == 0

    mesh = plsc.VectorSubcoreMesh(core_axis_name="c", subcore_axis_name="s")
    params = pltpu.CompilerParams(use_tc_tiling_on_sc=True,
                                  needs_layout_passes=False)

    tok_t = tokens_ids.T
    table_t = table.T
    tail = jnp.reshape(table[vocab_main:, :], (tail_rows, 2 * EMBED))

    pairs_kernel = pl.kernel(
        functools.partial(_pairs_body, vocab_main=vocab_main),
        out_type=jax.ShapeDtypeStruct((vocab // 2, 2 * EMBED), jnp.float32),
        mesh=mesh,
        scratch_types=[
            pltpu.VMEM((NSLOT, EMBED, VBLK), jnp.float32),
            pltpu.VMEM((NSLOT, VBLK // 2, 2 * EMBED), jnp.float32),
            pltpu.VMEM((tail_rows, 2 * EMBED), jnp.float32),
            pltpu.SemaphoreType.DMA,
            pltpu.SemaphoreType.DMA,
            pltpu.SemaphoreType.DMA,
            pltpu.SemaphoreType.DMA,
        ],
        compiler_params=params,
    )
    table_pairs = pairs_kernel(table_t, tail)

    emb_kernel = pl.kernel(
        functools.partial(_emb_body, seq=seq),
        out_type=jax.ShapeDtypeStruct((seq, embed, batch), jnp.float32),
        mesh=mesh,
        scratch_types=[
            pltpu.VMEM((seq, BCOLS), jnp.int32),
            pltpu.VMEM((seq, BCOLS), jnp.int32),
            pltpu.VMEM((seq, BCOLS), jnp.int32),
            pltpu.VMEM((NSLOT, BCOLS, 2 * EMBED), jnp.float32),
            pltpu.VMEM((NSLOT, EMBED, BCOLS), jnp.float32),
            pltpu.SemaphoreType.DMA,
            pltpu.SemaphoreType.DMA,
            pltpu.SemaphoreType.DMA,
            pltpu.SemaphoreType.DMA,
        ],
        compiler_params=params,
    )
    out_t = emb_kernel(tok_t, table_pairs)
    return jnp.transpose(out_t, (2, 0, 1))
